# baseline (speedup 1.0000x reference)
import functools

import jax
import jax.numpy as jnp
from jax import lax
from jax.experimental import pallas as pl
from jax.experimental.pallas import tpu as pltpu
from jax.experimental.pallas import tpu_sc as plsc

N = 10000
E = 320000
D = 128
DH = D // 2
PAD = 8
NC = 2
NS = 16
NW = NC * NS
E_PER = E // NW
CH = 125
E_TILE = E // NS
NIT = E_TILE // CH
NBUF = 5
RP = 624
TAIL0 = NS * RP
TAILN = N - TAIL0


def _deg_body(row_hbm, col_hbm, out_hbm, row_v, col_v, deg_v):
    c = lax.axis_index("c")
    s = lax.axis_index("s")
    wid = s * NC + c
    base = wid * E_PER
    pltpu.sync_copy(row_hbm.at[pl.ds(base, E_PER)], row_v)
    pltpu.sync_copy(col_hbm.at[pl.ds(base, E_PER)], col_v)

    zeros16 = jnp.zeros((16,), jnp.float32)

    def zero_body(i, carry):
        deg_v[pl.ds(i * 16, 16)] = zeros16
        return carry

    lax.fori_loop(0, N // 16, zero_body, 0)

    ones16 = jnp.ones((16,), jnp.float32)

    def body(i, carry):
        r = row_v[pl.ds(i * 16, 16)]
        cc = col_v[pl.ds(i * 16, 16)]
        plsc.addupdate_scatter(deg_v, [r], ones16, mask=r != cc)
        return carry

    lax.fori_loop(0, E_PER // 16, body, 0)
    pltpu.sync_copy(deg_v, out_hbm.at[wid])


def _prop_body(xs_hbm, row_hbm, col_hbm, zeros_hbm, out_hbm,
               row_v, col_v, gbufs, accum, gsems, ssems):
    c = lax.axis_index("c")
    s = lax.axis_index("s")
    pltpu.sync_copy(zeros_hbm.at[pl.ds(s * RP, RP)],
                    accum.at[pl.ds(s * RP, RP)])

    @pl.when(s == 0)
    def _zero_tail():
        pltpu.sync_copy(zeros_hbm.at[pl.ds(TAIL0, TAILN)],
                        accum.at[pl.ds(TAIL0, TAILN)])

    pltpu.sync_copy(row_hbm.at[s], row_v)
    pltpu.sync_copy(col_hbm.at[s], col_v)
    plsc.subcore_barrier()

    xs_c = xs_hbm.at[c]

    for b in range(NBUF):
        pltpu.async_copy(xs_c.at[row_v.at[b]], gbufs[b], gsems[b])

    def body(j, carry):
        for b in range(NBUF):
            i = NBUF * j + b
            pltpu.make_async_copy(xs_c.at[row_v.at[i]],
                                  gbufs[b], gsems[b]).wait()
            pltpu.async_copy(gbufs[b], accum.at[col_v.at[i]],
                             ssems[b], add=True)
        for b in range(NBUF):
            i = NBUF * j + b

            @pl.when(i + NBUF < NIT)
            def _next():
                pltpu.make_async_copy(gbufs[b], accum.at[col_v.at[i]],
                                      ssems[b]).wait()
                pltpu.async_copy(xs_c.at[row_v.at[i + NBUF]],
                                 gbufs[b], gsems[b])

        return carry

    lax.fori_loop(0, NIT // NBUF, body, 0)
    for b in range(NBUF):
        i = NIT - NBUF + b
        pltpu.make_async_copy(gbufs[b], accum.at[col_v.at[i]],
                              ssems[b]).wait()
    plsc.subcore_barrier()
    pltpu.sync_copy(accum.at[pl.ds(s * RP, RP)],
                    out_hbm.at[c, pl.ds(s * RP, RP)])

    @pl.when(s == 0)
    def _read_tail():
        pltpu.sync_copy(accum.at[pl.ds(TAIL0, TAILN)],
                        out_hbm.at[c, pl.ds(TAIL0, TAILN)])


@functools.lru_cache(maxsize=None)
def _sc_kernels():
    mesh = plsc.VectorSubcoreMesh(core_axis_name="c", subcore_axis_name="s",
                                  num_cores=NC, num_subcores=NS)
    deg = pl.kernel(
        _deg_body,
        out_type=jax.ShapeDtypeStruct((NW, N), jnp.float32),
        mesh=mesh,
        compiler_params=pltpu.CompilerParams(needs_layout_passes=False),
        scratch_types=[
            pltpu.VMEM((E_PER,), jnp.int32),
            pltpu.VMEM((E_PER,), jnp.int32),
            pltpu.VMEM((N,), jnp.float32),
        ],
    )
    prop = pl.kernel(
        _prop_body,
        out_type=jax.ShapeDtypeStruct((NC, N, DH), jnp.float32),
        mesh=mesh,
        compiler_params=pltpu.CompilerParams(use_tc_tiling_on_sc=False),
        scratch_types=[
            pltpu.VMEM((NIT, CH), jnp.int32),
            pltpu.VMEM((NIT, CH), jnp.int32),
            [pltpu.VMEM((CH, DH), jnp.float32)] * NBUF,
            pltpu.VMEM_SHARED((N, DH), jnp.float32),
            [pltpu.SemaphoreType.DMA] * NBUF,
            [pltpu.SemaphoreType.DMA] * NBUF,
        ],
    )
    return deg, prop


def _scale_body(degp, x, r, cc, xs, dinv, radj):
    deg = jnp.sum(degp[...], axis=0)
    di = jnp.where(deg > 0, lax.rsqrt(deg), 0.0)
    dinv[...] = di[:, None]
    scaled = x[...] * di[:, None]
    xs[0, :N, :] = scaled[:, :DH]
    xs[1, :N, :] = scaled[:, DH:]
    xs[0, N:, :] = jnp.zeros((PAD, DH), jnp.float32)
    xs[1, N:, :] = jnp.zeros((PAD, DH), jnp.float32)
    rr = r[...]
    radj[...] = jnp.where(rr == cc[...], N, rr)


_scale_call = pl.pallas_call(
    _scale_body,
    out_shape=[
        jax.ShapeDtypeStruct((NC, N + PAD, DH), jnp.float32),
        jax.ShapeDtypeStruct((N, 1), jnp.float32),
        jax.ShapeDtypeStruct((2500, 128), jnp.int32),
    ],
)


def _mid_body(p, dinv, tx1, xs1):
    di = dinv[...]
    t = -di * jnp.concatenate((p[0], p[1]), axis=1)
    tx1[...] = t
    scaled = di * t
    xs1[0, :N, :] = scaled[:, :DH]
    xs1[1, :N, :] = scaled[:, DH:]
    xs1[0, N:, :] = jnp.zeros((PAD, DH), jnp.float32)
    xs1[1, N:, :] = jnp.zeros((PAD, DH), jnp.float32)


_mid_call = pl.pallas_call(
    _mid_body,
    out_shape=[
        jax.ShapeDtypeStruct((N, D), jnp.float32),
        jax.ShapeDtypeStruct((NC, N + PAD, DH), jnp.float32),
    ],
)


def _mm_body(t, w, acc, out):
    out[...] = acc[...] + jnp.dot(t[...], w[...],
                                  preferred_element_type=jnp.float32)


_mm_call = pl.pallas_call(
    _mm_body, out_shape=jax.ShapeDtypeStruct((N, D), jnp.float32))


def _final_body(x, z01, p2, dinv, w2, b, a, g, be, out):
    xv = x[...]
    s2 = jnp.concatenate((p2[0], p2[1]), axis=1)
    tx2 = -2.0 * dinv[...] * s2 - xv
    z = z01[...] + jnp.dot(tx2, w2[...], preferred_element_type=jnp.float32)
    z = z + b[...]
    z = jnp.where(z >= 0, z, a[0, 0] * z)
    mean = jnp.mean(z, axis=0, keepdims=True)
    zc = z - mean
    var = jnp.mean(zc * zc, axis=0, keepdims=True)
    out[...] = zc * lax.rsqrt(var + 1e-5) * g[...] + be[...]


_final_call = pl.pallas_call(
    _final_body,
    out_shape=jax.ShapeDtypeStruct((N, D), jnp.float32),
)


def kernel(x, edge_index, W0, W1, W2, bias, prelu_a, bn_gamma, bn_beta):
    ei = edge_index.astype(jnp.int32)
    row = ei[0]
    col = ei[1]
    deg_kernel, prop_kernel = _sc_kernels()
    degp = deg_kernel(row, col)
    xs0, dinv, radj2 = _scale_call(degp, x, row.reshape(2500, 128),
                                   col.reshape(2500, 128))
    radj3 = radj2.reshape(NS, NIT, CH)
    col3 = col.reshape(NS, NIT, CH)
    zeros = jnp.zeros((N, DH), jnp.float32)
    p1 = prop_kernel(xs0, radj3, col3, zeros)
    z0 = _mm_call(x, W0, jnp.zeros((N, D), jnp.float32))
    tx1, xs1 = _mid_call(p1, dinv)
    p2 = prop_kernel(xs1, radj3, col3, zeros)
    z01 = _mm_call(tx1, W1, z0)
    out = _final_call(x, z01, p2, dinv, W2,
                      bias.reshape(1, D), prelu_a.reshape(1, 1),
                      bn_gamma.reshape(1, D), bn_beta.reshape(1, D))
    return out


# speedup vs baseline: 1.0190x; 1.0190x over previous
import functools

import jax
import jax.numpy as jnp
from jax import lax
from jax.experimental import pallas as pl
from jax.experimental.pallas import tpu as pltpu
from jax.experimental.pallas import tpu_sc as plsc

N = 10000
E = 320000
D = 128
DH = D // 2
PAD = 8
NC = 2
NS = 16
NW = NC * NS
E_PER = E // NW
CH = 125
E_TILE = E // NS
NIT = E_TILE // CH
NBUF = 5
RP = 624
TAIL0 = NS * RP
TAILN = N - TAIL0


def _deg_body(row_hbm, col_hbm, out_hbm, row_v, col_v, deg_v):
    c = lax.axis_index("c")
    s = lax.axis_index("s")
    wid = s * NC + c
    base = wid * E_PER
    pltpu.sync_copy(row_hbm.at[pl.ds(base, E_PER)], row_v)
    pltpu.sync_copy(col_hbm.at[pl.ds(base, E_PER)], col_v)

    zeros16 = jnp.zeros((16,), jnp.float32)

    def zero_body(i, carry):
        deg_v[pl.ds(i * 16, 16)] = zeros16
        return carry

    lax.fori_loop(0, N // 16, zero_body, 0)

    ones16 = jnp.ones((16,), jnp.float32)

    def body(i, carry):
        r = row_v[pl.ds(i * 16, 16)]
        cc = col_v[pl.ds(i * 16, 16)]
        plsc.addupdate_scatter(deg_v, [r], ones16, mask=r != cc)
        return carry

    lax.fori_loop(0, E_PER // 16, body, 0)
    pltpu.sync_copy(deg_v, out_hbm.at[wid])


def _prop_body(xs_hbm, row_hbm, col_hbm, zeros_hbm, out_hbm,
               row_v, col_v, gbufs, accum, gsems, ssems):
    c = lax.axis_index("c")
    s = lax.axis_index("s")
    pltpu.sync_copy(zeros_hbm.at[pl.ds(s * RP, RP)],
                    accum.at[pl.ds(s * RP, RP)])

    @pl.when(s == 0)
    def _zero_tail():
        pltpu.sync_copy(zeros_hbm.at[pl.ds(TAIL0, TAILN)],
                        accum.at[pl.ds(TAIL0, TAILN)])

    pltpu.sync_copy(row_hbm.at[s], row_v)
    pltpu.sync_copy(col_hbm.at[s], col_v)
    plsc.subcore_barrier()

    xs_c = xs_hbm.at[c]

    for b in range(NBUF):
        pltpu.async_copy(xs_c.at[row_v.at[b]], gbufs[b], gsems[b])

    def body(j, carry):
        for b in range(NBUF):
            i = NBUF * j + b
            pltpu.make_async_copy(xs_c.at[row_v.at[i]],
                                  gbufs[b], gsems[b]).wait()
            pltpu.async_copy(gbufs[b], accum.at[col_v.at[i]],
                             ssems[b], add=True)
        for b in range(NBUF):
            i = NBUF * j + b

            @pl.when(i + NBUF < NIT)
            def _next():
                pltpu.make_async_copy(gbufs[b], accum.at[col_v.at[i]],
                                      ssems[b]).wait()
                pltpu.async_copy(xs_c.at[row_v.at[i + NBUF]],
                                 gbufs[b], gsems[b])

        return carry

    lax.fori_loop(0, NIT // NBUF, body, 0)
    for b in range(NBUF):
        i = NIT - NBUF + b
        pltpu.make_async_copy(gbufs[b], accum.at[col_v.at[i]],
                              ssems[b]).wait()
    plsc.subcore_barrier()
    pltpu.sync_copy(accum.at[pl.ds(s * RP, RP)],
                    out_hbm.at[c, pl.ds(s * RP, RP)])

    @pl.when(s == 0)
    def _read_tail():
        pltpu.sync_copy(accum.at[pl.ds(TAIL0, TAILN)],
                        out_hbm.at[c, pl.ds(TAIL0, TAILN)])


@functools.lru_cache(maxsize=None)
def _sc_kernels():
    mesh = plsc.VectorSubcoreMesh(core_axis_name="c", subcore_axis_name="s",
                                  num_cores=NC, num_subcores=NS)
    deg = pl.kernel(
        _deg_body,
        out_type=jax.ShapeDtypeStruct((NW, N), jnp.float32),
        mesh=mesh,
        compiler_params=pltpu.CompilerParams(needs_layout_passes=False),
        scratch_types=[
            pltpu.VMEM((E_PER,), jnp.int32),
            pltpu.VMEM((E_PER,), jnp.int32),
            pltpu.VMEM((N,), jnp.float32),
        ],
    )
    prop = pl.kernel(
        _prop_body,
        out_type=jax.ShapeDtypeStruct((NC, N, DH), jnp.float32),
        mesh=mesh,
        compiler_params=pltpu.CompilerParams(use_tc_tiling_on_sc=False),
        scratch_types=[
            pltpu.VMEM((NIT, CH), jnp.int32),
            pltpu.VMEM((NIT, CH), jnp.int32),
            [pltpu.VMEM((CH, DH), jnp.float32)] * NBUF,
            pltpu.VMEM_SHARED((N, DH), jnp.float32),
            [pltpu.SemaphoreType.DMA] * NBUF,
            [pltpu.SemaphoreType.DMA] * NBUF,
        ],
    )
    return deg, prop


def _scale_body(degp, x, xs, dinv, m, sdeg):
    deg = jnp.sum(degp[...], axis=0)
    di = jnp.where(deg > 0, lax.rsqrt(deg), 0.0)
    dinv[...] = di[:, None]
    m[...] = -(di * di)[:, None]
    sdeg[...] = jnp.sqrt(deg)[:, None]
    scaled = x[...] * di[:, None]
    xs[0, :N, :] = scaled[:, :DH]
    xs[1, :N, :] = scaled[:, DH:]
    xs[0, N:, :] = jnp.zeros((PAD, DH), jnp.float32)
    xs[1, N:, :] = jnp.zeros((PAD, DH), jnp.float32)


_scale_call = pl.pallas_call(
    _scale_body,
    out_shape=[
        jax.ShapeDtypeStruct((NC, N + PAD, DH), jnp.float32),
        jax.ShapeDtypeStruct((N, 1), jnp.float32),
        jax.ShapeDtypeStruct((N, 1), jnp.float32),
        jax.ShapeDtypeStruct((N, 1), jnp.float32),
    ],
)


def _radj_body(r, cc, radj):
    rr = r[...]
    radj[...] = jnp.where(rr == cc[...], N, rr)


_radj_call = pl.pallas_call(
    _radj_body, out_shape=jax.ShapeDtypeStruct((2500, 128), jnp.int32))


def _mid_body(p, m, xs1):
    mv = m[...]
    xs1[0, :N, :] = p[0] * mv
    xs1[1, :N, :] = p[1] * mv
    xs1[0, N:, :] = jnp.zeros((PAD, DH), jnp.float32)
    xs1[1, N:, :] = jnp.zeros((PAD, DH), jnp.float32)


_mid_call = pl.pallas_call(
    _mid_body,
    out_shape=jax.ShapeDtypeStruct((NC, N + PAD, DH), jnp.float32),
)


def _mm0_body(x, w0, z0):
    z0[...] = jnp.dot(x[...], w0[...], preferred_element_type=jnp.float32)


_mm0_call = pl.pallas_call(
    _mm0_body, out_shape=jax.ShapeDtypeStruct((N, D), jnp.float32))


def _mm1_body(xs1, sdeg, w1, z0, z01):
    tx1 = jnp.concatenate((xs1[0, :N, :], xs1[1, :N, :]), axis=1) * sdeg[...]
    z01[...] = z0[...] + jnp.dot(tx1, w1[...],
                                 preferred_element_type=jnp.float32)


_mm1_call = pl.pallas_call(
    _mm1_body, out_shape=jax.ShapeDtypeStruct((N, D), jnp.float32))


def _final_body(x, z01, p2, dinv, w2, b, a, g, be, out):
    xv = x[...]
    s2 = jnp.concatenate((p2[0], p2[1]), axis=1)
    tx2 = -2.0 * dinv[...] * s2 - xv
    z = z01[...] + jnp.dot(tx2, w2[...], preferred_element_type=jnp.float32)
    z = z + b[...]
    z = jnp.where(z >= 0, z, a[0, 0] * z)
    mean = jnp.mean(z, axis=0, keepdims=True)
    zc = z - mean
    var = jnp.mean(zc * zc, axis=0, keepdims=True)
    out[...] = zc * lax.rsqrt(var + 1e-5) * g[...] + be[...]


_final_call = pl.pallas_call(
    _final_body,
    out_shape=jax.ShapeDtypeStruct((N, D), jnp.float32),
)


def kernel(x, edge_index, W0, W1, W2, bias, prelu_a, bn_gamma, bn_beta):
    ei = edge_index.astype(jnp.int32)
    row = ei[0]
    col = ei[1]
    deg_kernel, prop_kernel = _sc_kernels()
    degp = deg_kernel(row, col)
    radj2 = _radj_call(row.reshape(2500, 128), col.reshape(2500, 128))
    xs0, dinv, m, sdeg = _scale_call(degp, x)
    radj3 = radj2.reshape(NS, NIT, CH)
    col3 = col.reshape(NS, NIT, CH)
    zeros = jnp.zeros((N, DH), jnp.float32)
    p1 = prop_kernel(xs0, radj3, col3, zeros)
    z0 = _mm0_call(x, W0)
    xs1 = _mid_call(p1, m)
    p2 = prop_kernel(xs1, radj3, col3, zeros)
    z01 = _mm1_call(xs1, sdeg, W1, z0)
    out = _final_call(x, z01, p2, dinv, W2,
                      bias.reshape(1, D), prelu_a.reshape(1, 1),
                      bn_gamma.reshape(1, D), bn_beta.reshape(1, D))
    return out
